# baseline (speedup 1.0000x reference)
import jax
import jax.numpy as jnp
from jax import lax
from jax.experimental import pallas as pl
from jax.experimental.pallas import tpu as pltpu
from jax.experimental.pallas import tpu_sc as plsc

N = 1000000
BATCH = 16384
D = 64
BLK = 32768
GRID = (N + BLK - 1) // BLK

NC = 2
NS = 16
L = 16
NW = NC * NS
BPW = BATCH // NW


def _sweep_body(wt_ref, b_ref, ut_ref, it_ref, pu_ref, pi_ref):
    wu = wt_ref[0:D, :]
    wi = wt_ref[D:2 * D, :]
    pu_ref[...] = jnp.sum(ut_ref[...] * wu, axis=0) + b_ref[0]
    pi_ref[...] = jnp.sum(it_ref[...] * wi, axis=0)


_SWEEP = pl.pallas_call(
    _sweep_body,
    grid=(GRID,),
    in_specs=[
        pl.BlockSpec((2 * D, 1), lambda i: (0, 0)),
        pl.BlockSpec(memory_space=pltpu.SMEM),
        pl.BlockSpec((D, BLK), lambda i: (0, i)),
        pl.BlockSpec((D, BLK), lambda i: (0, i)),
    ],
    out_specs=[
        pl.BlockSpec((BLK,), lambda i: (i,)),
        pl.BlockSpec((BLK,), lambda i: (i,)),
    ],
    out_shape=[jax.ShapeDtypeStruct((N,), jnp.float32)] * 2,
    compiler_params=pltpu.CompilerParams(dimension_semantics=("parallel",)),
)


def _gather_body(user_hbm, item_hbm, pu_hbm, pi_hbm, out_hbm,
                 idx_u, idx_i, val_u, val_i, out_v, sem_u, sem_i):
    wid = lax.axis_index("s") * NC + lax.axis_index("c")
    base = wid * BPW
    cu = pltpu.async_copy(user_hbm.at[pl.ds(base, BPW)], idx_u, sem_u)
    ci = pltpu.async_copy(item_hbm.at[pl.ds(base, BPW)], idx_i, sem_i)
    cu.wait()
    ci.wait()
    gu = pltpu.async_copy(pu_hbm.at[idx_u], val_u, sem_u)
    gi = pltpu.async_copy(pi_hbm.at[idx_i], val_i, sem_i)
    gu.wait()
    gi.wait()
    for k in range(BPW // L):
        out_v[pl.ds(k * L, L)] = (val_u[pl.ds(k * L, L)] + val_i[pl.ds(k * L, L)])
    pltpu.sync_copy(out_v, out_hbm.at[pl.ds(base, BPW)])


_GATHER = pl.kernel(
    _gather_body,
    out_type=jax.ShapeDtypeStruct((BATCH,), jnp.float32),
    mesh=plsc.VectorSubcoreMesh(core_axis_name="c", subcore_axis_name="s"),
    compiler_params=pltpu.CompilerParams(needs_layout_passes=False,
                                         use_tc_tiling_on_sc=False),
    scratch_types=[
        pltpu.VMEM((BPW,), jnp.int32),
        pltpu.VMEM((BPW,), jnp.int32),
        pltpu.VMEM((BPW,), jnp.float32),
        pltpu.VMEM((BPW,), jnp.float32),
        pltpu.VMEM((BPW,), jnp.float32),
        pltpu.SemaphoreType.DMA,
        pltpu.SemaphoreType.DMA,
    ],
)


def kernel(user, item, user_table, item_table, W, b):
    wt = W.reshape(2 * D, 1)
    p_u, p_i = _SWEEP(wt, b, user_table.T, item_table.T)
    return _GATHER(user.astype(jnp.int32), item.astype(jnp.int32), p_u, p_i)
